# baseline (speedup 1.0000x reference)
import functools
import numpy as np
import jax
import jax.numpy as jnp
from jax import lax
from jax.experimental import pallas as pl
from jax.experimental.pallas import tpu as pltpu
from jax.experimental.pallas import tpu_sc as plsc

_DEG_HIST = np.array([0] * 28 + [500, 1000, 1500, 2000, 2000, 1500, 1000, 500],
                     dtype=np.float64)
_AVG_DEG_LOG = float((np.log(np.arange(_DEG_HIST.size) + 1.0) * _DEG_HIST).sum()
                     / _DEG_HIST.sum())

D = 128
N_PAD = 10240
N_RANGES = 64
R_NODES = 160
EB = 128
NW = 32
_BIG = 3.0e38


def _sc_stats_body(q_hbm, src_hbm, dst_hbm, rp_hbm, stats_out,
                   idxb, dstb, rows, rpb, acc, sem):
    nc = 2
    wid = lax.axis_index("s") * nc + lax.axis_index("c")

    def do_range(rr, _):
        rid = wid * (N_RANGES // NW) + rr
        n0 = rid * R_NODES

        def init_row(l, _):
            zeros = jnp.zeros((16,), jnp.float32)
            pinf = jnp.full((16,), _BIG, jnp.float32)
            ninf = jnp.full((16,), -_BIG, jnp.float32)
            for c in range(D // 16):
                sl = pl.ds(c * 16, 16)
                acc[l, 0, sl] = zeros
                acc[l, 1, sl] = pinf
                acc[l, 2, sl] = ninf
                acc[l, 3, sl] = zeros
            return 0

        lax.fori_loop(0, R_NODES + 1, init_row, 0)

        pltpu.sync_copy(rp_hbm.at[pl.ds(n0, 176)], rpb.at[pl.ds(0, 176)])
        e0 = rpb[pl.ds(0, 16)][0]
        e1 = rpb[pl.ds(R_NODES, 16)][0]
        e0a = (e0 // 8) * 8
        nb = (e1 - e0a + (EB - 1)) // EB

        def do_batch(b, _):
            base = e0a + b * EB
            pltpu.sync_copy(src_hbm.at[pl.ds(base, EB)], idxb)
            pltpu.sync_copy(dst_hbm.at[pl.ds(base, EB)], dstb.at[pl.ds(0, EB)])
            pltpu.async_copy(q_hbm.at[idxb], rows, sem).wait()
            lo = jnp.maximum(e0, base)
            hi = jnp.minimum(e1, base + EB)

            def do_edge(e, _):
                r = e - base
                l = dstb[pl.ds(r, 16)][0] - n0
                for c in range(D // 16):
                    sl = pl.ds(c * 16, 16)
                    v = rows[r, sl]
                    acc[l, 0, sl] += v
                    acc[l, 1, sl] = jnp.minimum(acc[l, 1, sl], v)
                    acc[l, 2, sl] = jnp.maximum(acc[l, 2, sl], v)
                    acc[l, 3, sl] += v * v
                return 0

            lax.fori_loop(lo, hi, do_edge, 0)
            return 0

        lax.fori_loop(0, nb, do_batch, 0)

        pltpu.sync_copy(acc.at[pl.ds(0, R_NODES)],
                        stats_out.at[pl.ds(n0, R_NODES)])
        return 0

    lax.fori_loop(0, N_RANGES // NW, do_range, 0)


def _sc_stats(q, src_pad, dst_pad, rp_pad):
    mesh = plsc.VectorSubcoreMesh(core_axis_name="c", subcore_axis_name="s")
    f = functools.partial(
        pl.kernel,
        mesh=mesh,
        out_type=jax.ShapeDtypeStruct((N_PAD, 4, D), jnp.float32),
        scratch_types=[
            pltpu.VMEM((EB,), jnp.int32),
            pltpu.VMEM((EB + 16,), jnp.int32),
            pltpu.VMEM((EB, D), jnp.float32),
            pltpu.VMEM((176 + 16,), jnp.int32),
            pltpu.VMEM((R_NODES + 1, 4, D), jnp.float32),
            pltpu.SemaphoreType.DMA,
        ],
    )(_sc_stats_body)
    return f(q, src_pad, dst_pad, rp_pad)


def _pq_body(o_ref, w1_ref, w2_ref, pb_ref, p_ref, q_ref):
    o = o_ref[...]
    p_ref[...] = jnp.dot(o, w1_ref[...],
                         preferred_element_type=jnp.float32) + pb_ref[...]
    q_ref[...] = jnp.dot(o, w2_ref[...], preferred_element_type=jnp.float32)


def _pq(o, w1, w2, preb):
    n = o.shape[0]
    blk = 400
    grid = n // blk
    return pl.pallas_call(
        _pq_body,
        grid=(grid,),
        in_specs=[
            pl.BlockSpec((blk, D), lambda i: (i, 0)),
            pl.BlockSpec((D, D), lambda i: (0, 0)),
            pl.BlockSpec((D, D), lambda i: (0, 0)),
            pl.BlockSpec((1, D), lambda i: (0, 0)),
        ],
        out_specs=[
            pl.BlockSpec((blk, D), lambda i: (i, 0)),
            pl.BlockSpec((blk, D), lambda i: (i, 0)),
        ],
        out_shape=[
            jax.ShapeDtypeStruct((n, D), jnp.float32),
            jax.ShapeDtypeStruct((n, D), jnp.float32),
        ],
    )(o, w1, w2, preb.reshape(1, D))


def _post_body(o_ref, p_ref, s_ref, mn_ref, mx_ref, s2_ref, rpa_ref, rpb_ref,
               w13_ref, pb_ref, lw_ref, lb_ref, out_ref):
    o = o_ref[...]
    p = p_ref[...]
    s = s_ref[...]
    cnt = (rpb_ref[...] - rpa_ref[...]).astype(jnp.float32)
    cntc = jnp.maximum(cnt, 1.0)
    has = cnt > 0.0
    sm = s / cntc
    mean = jnp.where(has, p + sm, 0.0)
    mn = jnp.where(has, p + mn_ref[...], 0.0)
    mx = jnp.where(has, p + mx_ref[...], 0.0)
    std = jnp.sqrt(jnp.maximum(s2_ref[...] / cntc - sm * sm, 0.0) + 1e-5)
    lg = jnp.log(cntc + 1.0)
    sa = lg / _AVG_DEG_LOG
    st = _AVG_DEG_LOG / lg
    w13 = w13_ref[...]

    def mm(a, k):
        return jnp.dot(a, w13[k * D:(k + 1) * D, :],
                       preferred_element_type=jnp.float32)

    base = mm(o, 0) + mm(mean, 1) + mm(mn, 2) + mm(mx, 3) + mm(std, 4)
    amp = mm(mean, 5) + mm(mn, 6) + mm(mx, 7) + mm(std, 8)
    att = mm(mean, 9) + mm(mn, 10) + mm(mx, 11) + mm(std, 12)
    t = base + sa * amp + st * att + pb_ref[...]
    out_ref[...] = jnp.dot(t, lw_ref[...],
                           preferred_element_type=jnp.float32) + lb_ref[...]


def _post(o, p, s, mn, mx, s2, rpa, rpb, w13, postb, linw, linb):
    n = o.shape[0]
    blk = 400
    grid = n // blk
    row = lambda i: (i, 0)
    fixed = lambda i: (0, 0)
    return pl.pallas_call(
        _post_body,
        grid=(grid,),
        in_specs=[
            pl.BlockSpec((blk, D), row),
            pl.BlockSpec((blk, D), row),
            pl.BlockSpec((blk, D), row),
            pl.BlockSpec((blk, D), row),
            pl.BlockSpec((blk, D), row),
            pl.BlockSpec((blk, D), row),
            pl.BlockSpec((blk, 1), row),
            pl.BlockSpec((blk, 1), row),
            pl.BlockSpec((13 * D, D), fixed),
            pl.BlockSpec((1, D), fixed),
            pl.BlockSpec((D, D), fixed),
            pl.BlockSpec((1, D), fixed),
        ],
        out_specs=pl.BlockSpec((blk, D), row),
        out_shape=jax.ShapeDtypeStruct((n, D), jnp.float32),
    )(o, p, s, mn, mx, s2, rpa, rpb, w13, postb.reshape(1, D), linw,
      linb.reshape(1, D))


def _bn_relu_body(o_ref, g_ref, b_ref, out_ref):
    o = o_ref[...]
    m = jnp.mean(o, axis=0, keepdims=True)
    v = jnp.mean((o - m) * (o - m), axis=0, keepdims=True)
    out_ref[...] = jnp.maximum(
        (o - m) / jnp.sqrt(v + 1e-5) * g_ref[...] + b_ref[...], 0.0)


def _bn_relu(o, g, b):
    return pl.pallas_call(
        _bn_relu_body,
        out_shape=jax.ShapeDtypeStruct(o.shape, o.dtype),
    )(o, g.reshape(1, -1), b.reshape(1, -1))


def kernel(x, edge_index, params):
    n, d = x.shape
    e = edge_index.shape[1]
    dst_s, src_s = lax.sort((edge_index[1], edge_index[0]), num_keys=1)
    rowptr = jnp.searchsorted(
        dst_s, jnp.arange(N_PAD + 176, dtype=jnp.int32), side="left"
    ).astype(jnp.int32)
    src_pad = jnp.concatenate(
        [src_s, jnp.zeros((256,), jnp.int32)])
    dst_pad = jnp.concatenate(
        [dst_s, jnp.zeros((256,), jnp.int32)])

    rpa = rowptr[:n].reshape(n, 1)
    rpb = rowptr[1:n + 1].reshape(n, 1)
    o = x
    hv = [x]
    for (preW, preb, postW, postb, linW, linb, g, b) in params:
        p, q = _pq(o, preW[:D], preW[D:], preb)
        stats = _sc_stats(q, src_pad, dst_pad, rowptr)
        s = stats[:n, 0]
        mn = stats[:n, 1]
        mx = stats[:n, 2]
        s2 = stats[:n, 3]
        pre = _post(o, p, s, mn, mx, s2, rpa, rpb, postW, postb, linW, linb)
        o = _bn_relu(pre, g, b)
        hv.append(o)
    return jnp.concatenate(hv, axis=1)


# speedup vs baseline: 1.2210x; 1.2210x over previous
import functools
import numpy as np
import jax
import jax.numpy as jnp
from jax import lax
from jax.experimental import pallas as pl
from jax.experimental.pallas import tpu as pltpu
from jax.experimental.pallas import tpu_sc as plsc

_DEG_HIST = np.array([0] * 28 + [500, 1000, 1500, 2000, 2000, 1500, 1000, 500],
                     dtype=np.float64)
_AVG_DEG_LOG = float((np.log(np.arange(_DEG_HIST.size) + 1.0) * _DEG_HIST).sum()
                     / _DEG_HIST.sum())

D = 128
N_PAD = 10240
N_RANGES = 64
R_NODES = 160
EB = 128
NW = 32
_BIG = 3.0e38


def _sc_stats_body(q_hbm, src_hbm, dst_hbm, rp_hbm, stats_out,
                   idxb0, idxb1, dstb0, dstb1, rows0, rows1, rpb, acc,
                   rsem0, rsem1, dsem0, dsem1):
    idxb = (idxb0, idxb1)
    dstb = (dstb0, dstb1)
    rows = (rows0, rows1)
    rsem = (rsem0, rsem1)
    dsem = (dsem0, dsem1)
    nc = 2
    wid = lax.axis_index("s") * nc + lax.axis_index("c")

    def do_range(rr, _):
        rid = wid * (N_RANGES // NW) + rr
        n0 = rid * R_NODES

        def init_row(l, _):
            zeros = jnp.zeros((16,), jnp.float32)
            pinf = jnp.full((16,), _BIG, jnp.float32)
            ninf = jnp.full((16,), -_BIG, jnp.float32)
            for c in range(D // 16):
                sl = pl.ds(c * 16, 16)
                acc[l, 0, sl] = zeros
                acc[l, 1, sl] = pinf
                acc[l, 2, sl] = ninf
                acc[l, 3, sl] = zeros
            return 0

        lax.fori_loop(0, R_NODES + 1, init_row, 0)

        pltpu.sync_copy(rp_hbm.at[pl.ds(n0, 176)], rpb.at[pl.ds(0, 176)])
        e0 = rpb[pl.ds(0, 16)][0]
        e1 = rpb[pl.ds(R_NODES, 16)][0]
        e0a = (e0 // 8) * 8
        nb = (e1 - e0a + (EB - 1)) // EB

        def prefetch(i, s):
            @pl.when(i < nb)
            def _():
                base = e0a + i * EB
                pltpu.sync_copy(src_hbm.at[pl.ds(base, EB)], idxb[s])
                pltpu.make_async_copy(
                    dst_hbm.at[pl.ds(base, EB)],
                    dstb[s].at[pl.ds(0, EB)], dsem[s]).start()
                pltpu.make_async_copy(
                    q_hbm.at[idxb[s]], rows[s], rsem[s]).start()

        def process(i, s):
            @pl.when(i < nb)
            def _():
                base = e0a + i * EB
                pltpu.make_async_copy(
                    dst_hbm.at[pl.ds(base, EB)],
                    dstb[s].at[pl.ds(0, EB)], dsem[s]).wait()
                pltpu.make_async_copy(
                    q_hbm.at[idxb[s]], rows[s], rsem[s]).wait()
                lo = jnp.maximum(e0, base)
                hi = jnp.minimum(e1, base + EB)

                def do_edge(e, _):
                    r = e - base
                    l = dstb[s][pl.ds(r, 16)][0] - n0
                    for c in range(D // 16):
                        sl = pl.ds(c * 16, 16)
                        v = rows[s][r, sl]
                        plsc.addupdate(acc.at[l, 0, sl], v)
                        acc[l, 1, sl] = jnp.minimum(acc[l, 1, sl], v)
                        acc[l, 2, sl] = jnp.maximum(acc[l, 2, sl], v)
                        plsc.addupdate(acc.at[l, 3, sl], v * v)
                    return 0

                lax.fori_loop(lo, hi, do_edge, 0)

        prefetch(0, 0)

        def do_pair(k, _):
            i0 = 2 * k
            prefetch(i0 + 1, 1)
            process(i0, 0)
            prefetch(i0 + 2, 0)
            process(i0 + 1, 1)
            return 0

        lax.fori_loop(0, (nb + 1) // 2, do_pair, 0)

        pltpu.sync_copy(acc.at[pl.ds(0, R_NODES)],
                        stats_out.at[pl.ds(n0, R_NODES)])
        return 0

    lax.fori_loop(0, N_RANGES // NW, do_range, 0)


def _sc_stats(q, src_pad, dst_pad, rp_pad):
    mesh = plsc.VectorSubcoreMesh(core_axis_name="c", subcore_axis_name="s")
    f = functools.partial(
        pl.kernel,
        mesh=mesh,
        out_type=jax.ShapeDtypeStruct((N_PAD, 4, D), jnp.float32),
        scratch_types=[
            pltpu.VMEM((EB,), jnp.int32),
            pltpu.VMEM((EB,), jnp.int32),
            pltpu.VMEM((EB + 16,), jnp.int32),
            pltpu.VMEM((EB + 16,), jnp.int32),
            pltpu.VMEM((EB, D), jnp.float32),
            pltpu.VMEM((EB, D), jnp.float32),
            pltpu.VMEM((176 + 16,), jnp.int32),
            pltpu.VMEM((R_NODES + 1, 4, D), jnp.float32),
            pltpu.SemaphoreType.DMA,
            pltpu.SemaphoreType.DMA,
            pltpu.SemaphoreType.DMA,
            pltpu.SemaphoreType.DMA,
        ],
    )(_sc_stats_body)
    return f(q, src_pad, dst_pad, rp_pad)


def _pq_body(o_ref, w1_ref, w2_ref, pb_ref, p_ref, q_ref):
    o = o_ref[...]
    p_ref[...] = jnp.dot(o, w1_ref[...],
                         preferred_element_type=jnp.float32) + pb_ref[...]
    q_ref[...] = jnp.dot(o, w2_ref[...], preferred_element_type=jnp.float32)


def _pq(o, w1, w2, preb):
    n = o.shape[0]
    blk = 400
    grid = n // blk
    return pl.pallas_call(
        _pq_body,
        grid=(grid,),
        in_specs=[
            pl.BlockSpec((blk, D), lambda i: (i, 0)),
            pl.BlockSpec((D, D), lambda i: (0, 0)),
            pl.BlockSpec((D, D), lambda i: (0, 0)),
            pl.BlockSpec((1, D), lambda i: (0, 0)),
        ],
        out_specs=[
            pl.BlockSpec((blk, D), lambda i: (i, 0)),
            pl.BlockSpec((blk, D), lambda i: (i, 0)),
        ],
        out_shape=[
            jax.ShapeDtypeStruct((n, D), jnp.float32),
            jax.ShapeDtypeStruct((n, D), jnp.float32),
        ],
    )(o, w1, w2, preb.reshape(1, D))


def _post_body(o_ref, p_ref, s_ref, mn_ref, mx_ref, s2_ref, rpa_ref, rpb_ref,
               w13_ref, pb_ref, lw_ref, lb_ref, out_ref):
    o = o_ref[...]
    p = p_ref[...]
    s = s_ref[...]
    cnt = (rpb_ref[...] - rpa_ref[...]).astype(jnp.float32)
    cntc = jnp.maximum(cnt, 1.0)
    has = cnt > 0.0
    sm = s / cntc
    mean = jnp.where(has, p + sm, 0.0)
    mn = jnp.where(has, p + mn_ref[...], 0.0)
    mx = jnp.where(has, p + mx_ref[...], 0.0)
    std = jnp.sqrt(jnp.maximum(s2_ref[...] / cntc - sm * sm, 0.0) + 1e-5)
    lg = jnp.log(cntc + 1.0)
    sa = lg / _AVG_DEG_LOG
    st = _AVG_DEG_LOG / lg
    w13 = w13_ref[...]

    def mm(a, k):
        return jnp.dot(a, w13[k * D:(k + 1) * D, :],
                       preferred_element_type=jnp.float32)

    base = mm(o, 0) + mm(mean, 1) + mm(mn, 2) + mm(mx, 3) + mm(std, 4)
    amp = mm(mean, 5) + mm(mn, 6) + mm(mx, 7) + mm(std, 8)
    att = mm(mean, 9) + mm(mn, 10) + mm(mx, 11) + mm(std, 12)
    t = base + sa * amp + st * att + pb_ref[...]
    out_ref[...] = jnp.dot(t, lw_ref[...],
                           preferred_element_type=jnp.float32) + lb_ref[...]


def _post(o, p, s, mn, mx, s2, rpa, rpb, w13, postb, linw, linb):
    n = o.shape[0]
    blk = 400
    grid = n // blk
    row = lambda i: (i, 0)
    fixed = lambda i: (0, 0)
    return pl.pallas_call(
        _post_body,
        grid=(grid,),
        in_specs=[
            pl.BlockSpec((blk, D), row),
            pl.BlockSpec((blk, D), row),
            pl.BlockSpec((blk, D), row),
            pl.BlockSpec((blk, D), row),
            pl.BlockSpec((blk, D), row),
            pl.BlockSpec((blk, D), row),
            pl.BlockSpec((blk, 1), row),
            pl.BlockSpec((blk, 1), row),
            pl.BlockSpec((13 * D, D), fixed),
            pl.BlockSpec((1, D), fixed),
            pl.BlockSpec((D, D), fixed),
            pl.BlockSpec((1, D), fixed),
        ],
        out_specs=pl.BlockSpec((blk, D), row),
        out_shape=jax.ShapeDtypeStruct((n, D), jnp.float32),
    )(o, p, s, mn, mx, s2, rpa, rpb, w13, postb.reshape(1, D), linw,
      linb.reshape(1, D))


def _bn_relu_body(o_ref, g_ref, b_ref, out_ref):
    o = o_ref[...]
    m = jnp.mean(o, axis=0, keepdims=True)
    v = jnp.mean((o - m) * (o - m), axis=0, keepdims=True)
    out_ref[...] = jnp.maximum(
        (o - m) / jnp.sqrt(v + 1e-5) * g_ref[...] + b_ref[...], 0.0)


def _bn_relu(o, g, b):
    return pl.pallas_call(
        _bn_relu_body,
        out_shape=jax.ShapeDtypeStruct(o.shape, o.dtype),
    )(o, g.reshape(1, -1), b.reshape(1, -1))


def kernel(x, edge_index, params):
    n, d = x.shape
    e = edge_index.shape[1]
    dst_s, src_s = lax.sort((edge_index[1], edge_index[0]), num_keys=1)
    rowptr = jnp.searchsorted(
        dst_s, jnp.arange(N_PAD + 176, dtype=jnp.int32), side="left"
    ).astype(jnp.int32)
    src_pad = jnp.concatenate(
        [src_s, jnp.zeros((256,), jnp.int32)])
    dst_pad = jnp.concatenate(
        [dst_s, jnp.zeros((256,), jnp.int32)])

    rpa = rowptr[:n].reshape(n, 1)
    rpb = rowptr[1:n + 1].reshape(n, 1)
    o = x
    hv = [x]
    for (preW, preb, postW, postb, linW, linb, g, b) in params:
        p, q = _pq(o, preW[:D], preW[D:], preb)
        stats = _sc_stats(q, src_pad, dst_pad, rowptr)
        s = stats[:n, 0]
        mn = stats[:n, 1]
        mx = stats[:n, 2]
        s2 = stats[:n, 3]
        pre = _post(o, p, s, mn, mx, s2, rpa, rpb, postW, postb, linW, linb)
        o = _bn_relu(pre, g, b)
        hv.append(o)
    return jnp.concatenate(hv, axis=1)


# speedup vs baseline: 1.2426x; 1.0177x over previous
import functools
import numpy as np
import jax
import jax.numpy as jnp
from jax import lax
from jax.experimental import pallas as pl
from jax.experimental.pallas import tpu as pltpu
from jax.experimental.pallas import tpu_sc as plsc

_DEG_HIST = np.array([0] * 28 + [500, 1000, 1500, 2000, 2000, 1500, 1000, 500],
                     dtype=np.float64)
_AVG_DEG_LOG = float((np.log(np.arange(_DEG_HIST.size) + 1.0) * _DEG_HIST).sum()
                     / _DEG_HIST.sum())

D = 128
N_PAD = 10240
N_RANGES = 64
R_NODES = 160
EB = 128
NW = 32
_BIG = 3.0e38


def _sc_stats_body(q_hbm, src_hbm, dst_hbm, rp_hbm, stats_out,
                   idxb0, idxb1, dstb0, dstb1, rows0, rows1, rpb, acc,
                   rsem0, rsem1, dsem0, dsem1):
    idxb = (idxb0, idxb1)
    dstb = (dstb0, dstb1)
    rows = (rows0, rows1)
    rsem = (rsem0, rsem1)
    dsem = (dsem0, dsem1)
    nc = 2
    wid = lax.axis_index("s") * nc + lax.axis_index("c")

    def do_range(rr, _):
        rid = wid * (N_RANGES // NW) + rr
        n0 = rid * R_NODES

        def init_row(l, _):
            zeros = jnp.zeros((16,), jnp.float32)
            pinf = jnp.full((16,), _BIG, jnp.float32)
            ninf = jnp.full((16,), -_BIG, jnp.float32)
            for c in range(D // 16):
                sl = pl.ds(c * 16, 16)
                acc[l, 0, sl] = zeros
                acc[l, 1, sl] = pinf
                acc[l, 2, sl] = ninf
                acc[l, 3, sl] = zeros
            return 0

        lax.fori_loop(0, R_NODES + 1, init_row, 0)

        pltpu.sync_copy(rp_hbm.at[pl.ds(n0, 176)], rpb.at[pl.ds(0, 176)])
        e0 = rpb[pl.ds(0, 16)][0]
        e1 = rpb[pl.ds(R_NODES, 16)][0]
        e0a = (e0 // 8) * 8
        nb = (e1 - e0a + (EB - 1)) // EB

        def prefetch(i, s):
            @pl.when(i < nb)
            def _():
                base = e0a + i * EB
                pltpu.sync_copy(src_hbm.at[pl.ds(base, EB)], idxb[s])
                pltpu.make_async_copy(
                    dst_hbm.at[pl.ds(base, EB)],
                    dstb[s].at[pl.ds(0, EB)], dsem[s]).start()
                pltpu.make_async_copy(
                    q_hbm.at[idxb[s]], rows[s], rsem[s]).start()

        def process(i, s):
            @pl.when(i < nb)
            def _():
                base = e0a + i * EB
                pltpu.make_async_copy(
                    dst_hbm.at[pl.ds(base, EB)],
                    dstb[s].at[pl.ds(0, EB)], dsem[s]).wait()
                pltpu.make_async_copy(
                    q_hbm.at[idxb[s]], rows[s], rsem[s]).wait()
                lo = jnp.maximum(e0, base)
                hi = jnp.minimum(e1, base + EB)

                def do_edge(e, _):
                    r = e - base
                    l = dstb[s][pl.ds(r, 16)][0] - n0
                    for c in range(D // 16):
                        sl = pl.ds(c * 16, 16)
                        v = rows[s][r, sl]
                        plsc.addupdate(acc.at[l, 0, sl], v)
                        acc[l, 1, sl] = jnp.minimum(acc[l, 1, sl], v)
                        acc[l, 2, sl] = jnp.maximum(acc[l, 2, sl], v)
                        plsc.addupdate(acc.at[l, 3, sl], v * v)
                    return 0

                lax.fori_loop(lo, hi, do_edge, 0)

        prefetch(0, 0)

        def do_pair(k, _):
            i0 = 2 * k
            prefetch(i0 + 1, 1)
            process(i0, 0)
            prefetch(i0 + 2, 0)
            process(i0 + 1, 1)
            return 0

        lax.fori_loop(0, (nb + 1) // 2, do_pair, 0)

        pltpu.sync_copy(acc.at[pl.ds(0, R_NODES)],
                        stats_out.at[pl.ds(n0, R_NODES)])
        return 0

    lax.fori_loop(0, N_RANGES // NW, do_range, 0)


def _sc_stats(q, src_pad, dst_pad, rp_pad):
    mesh = plsc.VectorSubcoreMesh(core_axis_name="c", subcore_axis_name="s")
    f = functools.partial(
        pl.kernel,
        mesh=mesh,
        out_type=jax.ShapeDtypeStruct((N_PAD, 4, D), jnp.float32),
        scratch_types=[
            pltpu.VMEM((EB,), jnp.int32),
            pltpu.VMEM((EB,), jnp.int32),
            pltpu.VMEM((EB + 16,), jnp.int32),
            pltpu.VMEM((EB + 16,), jnp.int32),
            pltpu.VMEM((EB, D), jnp.float32),
            pltpu.VMEM((EB, D), jnp.float32),
            pltpu.VMEM((176 + 16,), jnp.int32),
            pltpu.VMEM((R_NODES + 1, 4, D), jnp.float32),
            pltpu.SemaphoreType.DMA,
            pltpu.SemaphoreType.DMA,
            pltpu.SemaphoreType.DMA,
            pltpu.SemaphoreType.DMA,
        ],
    )(_sc_stats_body)
    return f(q, src_pad, dst_pad, rp_pad)


def _pq_body(o_ref, w1_ref, w2_ref, pb_ref, p_ref, q_ref):
    o = o_ref[...]
    p_ref[...] = jnp.dot(o, w1_ref[...],
                         preferred_element_type=jnp.float32) + pb_ref[...]
    q_ref[...] = jnp.dot(o, w2_ref[...], preferred_element_type=jnp.float32)


def _pq(o, w1, w2, preb):
    n = o.shape[0]
    blk = 400
    grid = n // blk
    return pl.pallas_call(
        _pq_body,
        grid=(grid,),
        in_specs=[
            pl.BlockSpec((blk, D), lambda i: (i, 0)),
            pl.BlockSpec((D, D), lambda i: (0, 0)),
            pl.BlockSpec((D, D), lambda i: (0, 0)),
            pl.BlockSpec((1, D), lambda i: (0, 0)),
        ],
        out_specs=[
            pl.BlockSpec((blk, D), lambda i: (i, 0)),
            pl.BlockSpec((blk, D), lambda i: (i, 0)),
        ],
        out_shape=[
            jax.ShapeDtypeStruct((n, D), jnp.float32),
            jax.ShapeDtypeStruct((n, D), jnp.float32),
        ],
    )(o, w1, w2, preb.reshape(1, D))


def _post_body(o_ref, p_ref, st_ref, rpa_ref, rpb_ref,
               w13_ref, pb_ref, lw_ref, lb_ref, out_ref):
    o = o_ref[...]
    p = p_ref[...]
    s = st_ref[:, 0, :]
    cnt = (rpb_ref[...] - rpa_ref[...]).astype(jnp.float32)
    cntc = jnp.maximum(cnt, 1.0)
    has = cnt > 0.0
    sm = s / cntc
    mean = jnp.where(has, p + sm, 0.0)
    mn = jnp.where(has, p + st_ref[:, 1, :], 0.0)
    mx = jnp.where(has, p + st_ref[:, 2, :], 0.0)
    std = jnp.sqrt(jnp.maximum(st_ref[:, 3, :] / cntc - sm * sm, 0.0) + 1e-5)
    lg = jnp.log(cntc + 1.0)
    sa = lg / _AVG_DEG_LOG
    st = _AVG_DEG_LOG / lg
    w13 = w13_ref[...]

    def mm(a, k):
        return jnp.dot(a, w13[k * D:(k + 1) * D, :],
                       preferred_element_type=jnp.float32)

    t = (mm(o, 0) + mm(mean, 1) + mm(mn, 2) + mm(mx, 3) + mm(std, 4)
         + mm(sa * mean, 5) + mm(sa * mn, 6) + mm(sa * mx, 7) + mm(sa * std, 8)
         + mm(st * mean, 9) + mm(st * mn, 10) + mm(st * mx, 11)
         + mm(st * std, 12) + pb_ref[...])
    out_ref[...] = jnp.dot(t, lw_ref[...],
                           preferred_element_type=jnp.float32) + lb_ref[...]


def _post(o, p, stats, rpa, rpb, w13, postb, linw, linb):
    n = o.shape[0]
    blk = 400
    grid = n // blk
    row = lambda i: (i, 0)
    fixed = lambda i: (0, 0)
    return pl.pallas_call(
        _post_body,
        grid=(grid,),
        in_specs=[
            pl.BlockSpec((blk, D), row),
            pl.BlockSpec((blk, D), row),
            pl.BlockSpec((blk, 4, D), lambda i: (i, 0, 0)),
            pl.BlockSpec((blk, 1), row),
            pl.BlockSpec((blk, 1), row),
            pl.BlockSpec((13 * D, D), fixed),
            pl.BlockSpec((1, D), fixed),
            pl.BlockSpec((D, D), fixed),
            pl.BlockSpec((1, D), fixed),
        ],
        out_specs=pl.BlockSpec((blk, D), row),
        out_shape=jax.ShapeDtypeStruct((n, D), jnp.float32),
    )(o, p, stats, rpa, rpb, w13, postb.reshape(1, D), linw,
      linb.reshape(1, D))


def _bn_relu_body(o_ref, g_ref, b_ref, out_ref):
    o = o_ref[...]
    m = jnp.mean(o, axis=0, keepdims=True)
    v = jnp.mean((o - m) * (o - m), axis=0, keepdims=True)
    out_ref[...] = jnp.maximum(
        (o - m) / jnp.sqrt(v + 1e-5) * g_ref[...] + b_ref[...], 0.0)


def _bn_relu(o, g, b):
    return pl.pallas_call(
        _bn_relu_body,
        out_shape=jax.ShapeDtypeStruct(o.shape, o.dtype),
    )(o, g.reshape(1, -1), b.reshape(1, -1))


def kernel(x, edge_index, params):
    n, d = x.shape
    e = edge_index.shape[1]
    dst_s, src_s = lax.sort((edge_index[1], edge_index[0]), num_keys=1)
    rowptr = jnp.searchsorted(
        dst_s, jnp.arange(N_PAD + 176, dtype=jnp.int32), side="left"
    ).astype(jnp.int32)
    src_pad = jnp.concatenate(
        [src_s, jnp.zeros((256,), jnp.int32)])
    dst_pad = jnp.concatenate(
        [dst_s, jnp.zeros((256,), jnp.int32)])

    rpa = rowptr[:n].reshape(n, 1)
    rpb = rowptr[1:n + 1].reshape(n, 1)
    o = x
    hv = [x]
    for (preW, preb, postW, postb, linW, linb, g, b) in params:
        p, q = _pq(o, preW[:D], preW[D:], preb)
        stats = _sc_stats(q, src_pad, dst_pad, rowptr)
        pre = _post(o, p, stats, rpa, rpb, postW, postb, linW, linb)
        o = _bn_relu(pre, g, b)
        hv.append(o)
    return jnp.concatenate(hv, axis=1)
